# baseline (speedup 1.0000x reference)
import functools

import jax
import jax.numpy as jnp
from jax import lax
from jax.experimental import pallas as pl
from jax.experimental.pallas import tpu as pltpu
from jax.experimental.pallas import tpu_sc as plsc

_D = 768
_T = 32
_K = 5
_GPB = 64
_RB = _GPB * _T
_NEG = -1e9


def _fused(wo_ref, x_ref, w1_ref, b1_ref, w2_ref, b2_ref,
           filt_ref, sc_ref, attn_ref, keep_ref, idx_ref, fidx_ref):
    wo = wo_ref[...].reshape(_RB, _D)
    filt_ref[...] = wo_ref[...]
    h = jnp.tanh(
        jax.lax.dot_general(
            wo, w1_ref[...], (((1,), (1,)), ((), ())),
            preferred_element_type=jnp.float32) + b1_ref[...])
    sT = jax.lax.dot_general(
        w2_ref[...], h, (((1,), (1,)), ((), ())),
        preferred_element_type=jnp.float32)
    sCol = jnp.transpose(sT)
    rr = jax.lax.broadcasted_iota(jnp.int32, (_RB, _T), 0)
    ll = jax.lax.broadcasted_iota(jnp.int32, (_RB, _T), 1)
    sd = jnp.where((rr % _T) == ll, jnp.broadcast_to(sCol, (_RB, _T)), 0.0)
    s2 = jnp.sum(sd.reshape(_GPB, _T, _T), axis=1) + b2_ref[0, 0]
    pad = x_ref[...].reshape(_GPB, _T) == 0
    sm = jnp.where(pad, _NEG, s2)
    sc_ref[...] = sm.reshape(1, _GPB, _T)
    mx = jnp.max(sm, axis=1, keepdims=True)
    ex = jnp.exp(sm - mx)
    attn_ref[...] = (ex / jnp.sum(ex, axis=1, keepdims=True)).reshape(1, _GPB, _T)

    it = jax.lax.broadcasted_iota(jnp.int32, (_GPB, _T), 1)
    work = sm
    keep = jnp.zeros((_GPB, _T), jnp.float32)
    cols = []
    for _ in range(_K):
        mj = jnp.max(work, axis=1, keepdims=True)
        aj = jnp.min(jnp.where(work == mj, it, _T), axis=1, keepdims=True)
        hit = it == aj
        keep = jnp.where(hit, 1.0, keep)
        work = jnp.where(hit, -jnp.inf, work)
        cols.append(aj)
    idx_ref[...] = jnp.concatenate(cols, axis=1).reshape(1, _GPB, _K)
    keep_ref[...] = jnp.where(pad, 0.0, keep).reshape(1, _GPB, _T)

    idxcat = jnp.concatenate(cols, axis=1)
    fl4 = jnp.concatenate(
        [jnp.concatenate([idxcat[16 * w + g:16 * w + g + 1, :] for g in range(16)],
                         axis=1) for w in range(4)], axis=0)
    rr4 = jax.lax.broadcasted_iota(jnp.int32, (4, 16 * _K), 0)
    ll80 = jax.lax.broadcasted_iota(jnp.int32, (4, 16 * _K), 1)
    fidx = fl4 + pl.program_id(0) * _RB + (rr4 * 16 + ll80 // _K) * _T
    fidx_ref[...] = fidx.reshape(4, 1, 16 * _K)


_GW = 16
_NW = 32


def _sc_gather_body(table_hbm, fidx_hbm, out_hbm, idx_v, rows_v, sem):
    wid = lax.axis_index("s") * 2 + lax.axis_index("c")
    g0 = wid * _GW
    pltpu.sync_copy(fidx_hbm.at[wid, 0, pl.ds(0, _GW * _K)], idx_v)
    pltpu.async_copy(table_hbm.at[idx_v], rows_v, sem).wait()
    pltpu.sync_copy(rows_v, out_hbm.at[pl.ds(g0 * _K, _GW * _K)])


def _sc_gather(table, fidx):
    kfn = functools.partial(
        pl.kernel,
        mesh=plsc.VectorSubcoreMesh(core_axis_name="c", subcore_axis_name="s"),
        out_type=jax.ShapeDtypeStruct((_NW * _GW * _K, _D), jnp.float32),
        scratch_types=[
            pltpu.VMEM((_GW * _K,), jnp.int32),
            pltpu.VMEM((_GW * _K, _D), jnp.float32),
            pltpu.SemaphoreType.DMA,
        ],
    )(_sc_gather_body)
    return kfn(table, fidx)


def kernel(word_out, x, W1, b1, W2, b2):
    B, S, T, D = word_out.shape
    x3 = x.astype(jnp.int32)
    b1r = b1.reshape(1, D)
    b2r = b2.reshape(1, 1)

    filt, sc, attn, keep, idx, fidx = pl.pallas_call(
        _fused,
        grid=(B,),
        in_specs=[
            pl.BlockSpec((1, S, T, D), lambda i: (i, 0, 0, 0)),
            pl.BlockSpec((1, S, T), lambda i: (i, 0, 0)),
            pl.BlockSpec((D, D), lambda i: (0, 0)),
            pl.BlockSpec((1, D), lambda i: (0, 0)),
            pl.BlockSpec((1, D), lambda i: (0, 0)),
            pl.BlockSpec((1, 1), lambda i: (0, 0)),
        ],
        out_specs=[
            pl.BlockSpec((1, S, T, D), lambda i: (i, 0, 0, 0)),
            pl.BlockSpec((1, S, T), lambda i: (i, 0, 0)),
            pl.BlockSpec((1, S, T), lambda i: (i, 0, 0)),
            pl.BlockSpec((1, S, T), lambda i: (i, 0, 0)),
            pl.BlockSpec((1, S, _K), lambda i: (i, 0, 0)),
            pl.BlockSpec((4, 1, 16 * _K), lambda i: (i, 0, 0)),
        ],
        out_shape=[
            jax.ShapeDtypeStruct((B, S, T, D), jnp.float32),
            jax.ShapeDtypeStruct((B, S, T), jnp.float32),
            jax.ShapeDtypeStruct((B, S, T), jnp.float32),
            jax.ShapeDtypeStruct((B, S, T), jnp.float32),
            jax.ShapeDtypeStruct((B, S, _K), jnp.int32),
            jax.ShapeDtypeStruct((_NW, 1, 16 * _K), jnp.int32),
        ],
    )(word_out, x3, W1, b1r, W2, b2r)

    tv2 = _sc_gather(word_out.reshape(B * S * T, D), fidx)
    return (filt, sc, keep, attn, idx, tv2.reshape(B, S, _K, D))
